# baseline (speedup 1.0000x reference)
import functools

import jax
import jax.numpy as jnp
from jax import lax
from jax.experimental import pallas as pl
from jax.experimental.pallas import tpu as pltpu
from jax.experimental.pallas import tpu_sc as plsc

N = 10000
DF = 128
H = 8
C = 16
HC = H * C
NP = 10112
CH = 48
NWORK = 32
RPW = 216
E_PAD = NWORK * RPW * CH
NB = 640


def _proj_body(x_ref, w_ref, bsd_ref, t_ref, sd_ref):
    h = jnp.dot(x_ref[...], w_ref[...], preferred_element_type=jnp.float32)
    hb = jnp.dot(h, bsd_ref[...], preferred_element_type=jnp.float32)
    t_ref[:, :HC] = h
    t_ref[:, HC:] = hb
    sd_ref[...] = hb


_proj = pl.pallas_call(
    _proj_body,
    grid=(10,),
    in_specs=[
        pl.BlockSpec((N // 10, DF), lambda i: (i, 0)),
        pl.BlockSpec((DF, HC), lambda i: (0, 0)),
        pl.BlockSpec((DF, HC), lambda i: (0, 0)),
    ],
    out_specs=[
        pl.BlockSpec((N // 10, 2 * HC), lambda i: (i, 0)),
        pl.BlockSpec((N // 10, HC), lambda i: (i, 0)),
    ],
    out_shape=[
        jax.ShapeDtypeStruct((N, 2 * HC), jnp.float32),
        jax.ShapeDtypeStruct((N, HC), jnp.float32),
    ],
)


_mesh = plsc.VectorSubcoreMesh(core_axis_name="c", subcore_axis_name="s")


@functools.partial(
    pl.kernel,
    mesh=_mesh,
    out_type=(
        jax.ShapeDtypeStruct((2, NP, HC), jnp.float32),
        jax.ShapeDtypeStruct((2, NB, HC), jnp.float32),
    ),
    scratch_types=[
        pltpu.VMEM((2, CH), jnp.int32),
        pltpu.VMEM((2, CH), jnp.int32),
        pltpu.VMEM((CH,), jnp.int32),
        pltpu.VMEM((CH,), jnp.int32),
        pltpu.VMEM((CH,), jnp.int32),
        pltpu.VMEM((CH,), jnp.int32),
        pltpu.VMEM((CH, 2 * HC), jnp.float32),
        pltpu.VMEM((CH, HC), jnp.float32),
        pltpu.VMEM((CH, 2 * HC), jnp.float32),
        pltpu.VMEM((CH, HC), jnp.float32),
        pltpu.VMEM((CH, HC), jnp.float32),
        pltpu.VMEM_SHARED((NP, HC), jnp.float32),
        pltpu.VMEM_SHARED((NB, HC), jnp.float32),
        pltpu.SemaphoreType.DMA,
        pltpu.SemaphoreType.DMA,
        pltpu.SemaphoreType.DMA,
        pltpu.SemaphoreType.DMA,
    ],
)
def _edge_pass(t_hbm, sd_hbm, idx_hbm, za_hbm,
               msg_out, den_out,
               idx_a, idx_b, dsts_a, dsts_b, dstq_a, dstq_b,
               gt_a, gd_a, gt_b, gd_b, msg_v,
               acc, accb, sem_a, sem_b, semi_a, semi_b):
    cc = lax.axis_index("c")
    s = lax.axis_index("s")
    wid = s * 2 + cc
    rows_per_tile = NP // 16
    row0 = s * rows_per_tile
    brows_per_tile = NB // 16
    brow0 = s * brows_per_tile

    pltpu.sync_copy(za_hbm.at[pl.ds(row0, rows_per_tile)],
                    acc.at[pl.ds(row0, rows_per_tile)])
    pltpu.sync_copy(za_hbm.at[pl.ds(0, brows_per_tile)],
                    accb.at[pl.ds(brow0, brows_per_tile)])
    plsc.subcore_barrier()

    lane = lax.iota(jnp.int32, 16)
    z16 = jnp.zeros((16,), jnp.float32)

    def start_gathers(r, idx_v, gt_v, gd_v, sem):
        pltpu.async_copy(t_hbm.at[idx_v.at[0]], gt_v, sem)
        pltpu.async_copy(sd_hbm.at[idx_v.at[1]], gd_v, sem)

    def wait_gathers(idx_v, gt_v, gd_v, sem):
        pltpu.make_async_copy(t_hbm.at[idx_v.at[0]], gt_v, sem).wait()
        pltpu.make_async_copy(sd_hbm.at[idx_v.at[1]], gd_v, sem).wait()

    def half_round(r, r_next, guard, idx_v, dsts_v, dstq_v,
                   gt_v, gd_v, sem, semi):
        wait_gathers(idx_v, gt_v, gd_v, sem)
        @pl.when(guard)
        def _():
            pltpu.async_copy(idx_hbm.at[wid * RPW + r_next], idx_v, semi)
        for k in range(CH // 16):
            dv = idx_v[1, pl.ds(16 * k, 16)]
            dsts_v[pl.ds(16 * k, 16)] = dv
            dstq_v[pl.ds(16 * k, 16)] = lax.shift_right_logical(dv, 4)
        for k in range(CH // 16):
            dvec = dsts_v[pl.ds(16 * k, 16)]
            for t in range(16):
                e = 16 * k + t
                u = gt_v[e, pl.ds(HC, 16)] + gd_v[e, pl.ds(16, 16)]
                u = jnp.where(u >= 0.0, u, 0.2 * u)
                w = jnp.exp(u)
                for j in range(H):
                    msg_v[e, pl.ds(C * j, 16)] = (
                        gt_v[e, pl.ds(C * j, 16)] * w[j])
                gd_v[e, pl.ds(0, 16)] = z16
                gd_v[e, pl.ds(16, 16)] = z16
                m = dvec[t] & 15
                woff = jnp.where(m == 15, 112, m * H)
                mlo = jnp.where(lane < H, w, 0.0)
                fm = jnp.where(m == 15, 1.0, 0.0)
                val = mlo + (w - 2.0 * mlo) * fm
                gd_v[e, pl.ds(woff, 16)] = val
        pltpu.sync_copy(msg_v, acc.at[dsts_v], add=True)
        pltpu.sync_copy(gd_v, accb.at[dstq_v], add=True)
        @pl.when(guard)
        def _():
            pltpu.make_async_copy(idx_hbm.at[wid * RPW + r_next],
                                  idx_v, semi).wait()
            start_gathers(r_next, idx_v, gt_v, gd_v, sem)

    pltpu.sync_copy(idx_hbm.at[wid * RPW], idx_a)
    start_gathers(0, idx_a, gt_a, gd_a, sem_a)
    pltpu.sync_copy(idx_hbm.at[wid * RPW + 1], idx_b)
    start_gathers(1, idx_b, gt_b, gd_b, sem_b)

    def round_body(r2, carry):
        r = 2 * r2
        guard = r2 < RPW // 2 - 1
        half_round(r, r + 2, guard, idx_a, dsts_a, dstq_a,
                   gt_a, gd_a, sem_a, semi_a)
        half_round(r + 1, r + 3, guard, idx_b, dsts_b, dstq_b,
                   gt_b, gd_b, sem_b, semi_b)
        return carry

    lax.fori_loop(0, RPW // 2, round_body, 0)

    plsc.subcore_barrier()
    pltpu.sync_copy(acc.at[pl.ds(row0, rows_per_tile)],
                    msg_out.at[cc, pl.ds(row0, rows_per_tile)])
    pltpu.sync_copy(accb.at[pl.ds(brow0, brows_per_tile)],
                    den_out.at[cc, pl.ds(brow0, brows_per_tile)])


def _post_body(m0_ref, m1_ref, d0_ref, d1_ref, b_ref, g_ref, be_ref, o_ref):
    comb = m0_ref[...] + m1_ref[...]
    den = d0_ref[...] + d1_ref[...]
    jj = lax.broadcasted_iota(jnp.int32, (H, HC), 0)
    ll = lax.broadcasted_iota(jnp.int32, (H, HC), 1)
    expand = (ll // C == jj).astype(jnp.float32)
    denb = jnp.dot(den, expand, preferred_element_type=jnp.float32)
    o = comb / denb + b_ref[...]
    mean = jnp.mean(o, axis=0, keepdims=True)
    var = jnp.mean((o - mean) ** 2, axis=0, keepdims=True)
    o = (o - mean) * lax.rsqrt(var + 1e-5) * g_ref[...] + be_ref[...]
    o_ref[...] = jnp.where(o >= 0.0, o, 0.01 * o)


_post = pl.pallas_call(
    _post_body,
    in_specs=[
        pl.BlockSpec((N, HC), lambda: (0, 0)),
        pl.BlockSpec((N, HC), lambda: (0, 0)),
        pl.BlockSpec((N, H), lambda: (0, 0)),
        pl.BlockSpec((N, H), lambda: (0, 0)),
        pl.BlockSpec((1, HC), lambda: (0, 0)),
        pl.BlockSpec((1, HC), lambda: (0, 0)),
        pl.BlockSpec((1, HC), lambda: (0, 0)),
    ],
    out_specs=pl.BlockSpec((N, HC), lambda: (0, 0)),
    out_shape=jax.ShapeDtypeStruct((N, HC), jnp.float32),
)


def kernel(x, edge_idx, W, att_src, att_dst, bias, gamma, beta):
    e_real = edge_idx.shape[1]
    loops = jnp.arange(N, dtype=jnp.int32)
    n_pad = E_PAD - e_real - N
    src = jnp.concatenate([
        edge_idx[0].astype(jnp.int32), loops, jnp.zeros((n_pad,), jnp.int32)])
    dst = jnp.concatenate([
        edge_idx[1].astype(jnp.int32), loops, jnp.full((n_pad,), N, jnp.int32)])
    idx = jnp.stack([src.reshape(NWORK * RPW, CH),
                     dst.reshape(NWORK * RPW, CH)], axis=1)

    eye = jnp.eye(H, dtype=jnp.float32)
    ms = (att_src[:, :, None] * eye[:, None, :]).reshape(HC, H)
    md = (att_dst[:, :, None] * eye[:, None, :]).reshape(HC, H)
    bsd = jnp.concatenate(
        [ms, ms, md, md, jnp.zeros((HC, HC - 4 * H), jnp.float32)], axis=1)

    t, sd = _proj(x, W, bsd)
    za = jnp.zeros((NP, HC), jnp.float32)
    msg_p, den_p = _edge_pass(t, sd, idx, za)
    d2 = den_p.reshape(2, NB * 16, H)[:, :N]
    out = _post(msg_p[0, :N], msg_p[1, :N], d2[0], d2[1],
                bias.reshape(1, HC), gamma.reshape(1, HC), beta.reshape(1, HC))
    return out


# speedup vs baseline: 1.0425x; 1.0425x over previous
import functools

import jax
import jax.numpy as jnp
from jax import lax
from jax.experimental import pallas as pl
from jax.experimental.pallas import tpu as pltpu
from jax.experimental.pallas import tpu_sc as plsc

N = 10000
DF = 128
H = 8
C = 16
HC = H * C
NP = 10112
CH = 48
NWORK = 32
RPW = 216
E_PAD = NWORK * RPW * CH
NB = 640


def _proj_body(x_ref, w_ref, bsd_ref, t_ref, sd_ref):
    h = jnp.dot(x_ref[...], w_ref[...], preferred_element_type=jnp.float32)
    hb = jnp.dot(h, bsd_ref[...], preferred_element_type=jnp.float32)
    t_ref[:, :HC] = h
    t_ref[:, HC:] = hb
    sd_ref[:, :112] = jnp.zeros((x_ref.shape[0], 112), jnp.float32)
    sd_ref[:, 112:] = hb[:, 16:32]


_proj = pl.pallas_call(
    _proj_body,
    grid=(10,),
    in_specs=[
        pl.BlockSpec((N // 10, DF), lambda i: (i, 0)),
        pl.BlockSpec((DF, HC), lambda i: (0, 0)),
        pl.BlockSpec((DF, HC), lambda i: (0, 0)),
    ],
    out_specs=[
        pl.BlockSpec((N // 10, 2 * HC), lambda i: (i, 0)),
        pl.BlockSpec((N // 10, HC), lambda i: (i, 0)),
    ],
    out_shape=[
        jax.ShapeDtypeStruct((N, 2 * HC), jnp.float32),
        jax.ShapeDtypeStruct((N, HC), jnp.float32),
    ],
)


_mesh = plsc.VectorSubcoreMesh(core_axis_name="c", subcore_axis_name="s")


@functools.partial(
    pl.kernel,
    mesh=_mesh,
    out_type=(
        jax.ShapeDtypeStruct((2, NP, HC), jnp.float32),
        jax.ShapeDtypeStruct((2, NB, HC), jnp.float32),
    ),
    scratch_types=[
        pltpu.VMEM((2, CH), jnp.int32),
        pltpu.VMEM((2, CH), jnp.int32),
        pltpu.VMEM((CH,), jnp.int32),
        pltpu.VMEM((CH,), jnp.int32),
        pltpu.VMEM((CH,), jnp.int32),
        pltpu.VMEM((CH,), jnp.int32),
        pltpu.VMEM((CH, 2 * HC), jnp.float32),
        pltpu.VMEM((CH, HC), jnp.float32),
        pltpu.VMEM((CH, 2 * HC), jnp.float32),
        pltpu.VMEM((CH, HC), jnp.float32),
        pltpu.VMEM((CH, HC), jnp.float32),
        pltpu.VMEM_SHARED((NP, HC), jnp.float32),
        pltpu.VMEM_SHARED((NB, HC), jnp.float32),
        pltpu.SemaphoreType.DMA,
        pltpu.SemaphoreType.DMA,
        pltpu.SemaphoreType.DMA,
        pltpu.SemaphoreType.DMA,
    ],
)
def _edge_pass(t_hbm, sd_hbm, idx_hbm, za_hbm,
               msg_out, den_out,
               idx_a, idx_b, dsts_a, dsts_b, dstq_a, dstq_b,
               gt_a, gd_a, gt_b, gd_b, msg_v,
               acc, accb, sem_a, sem_b, semi_a, semi_b):
    cc = lax.axis_index("c")
    s = lax.axis_index("s")
    wid = s * 2 + cc
    rows_per_tile = NP // 16
    row0 = s * rows_per_tile
    brows_per_tile = NB // 16
    brow0 = s * brows_per_tile

    pltpu.sync_copy(za_hbm.at[pl.ds(row0, rows_per_tile)],
                    acc.at[pl.ds(row0, rows_per_tile)])
    pltpu.sync_copy(za_hbm.at[pl.ds(0, brows_per_tile)],
                    accb.at[pl.ds(brow0, brows_per_tile)])
    plsc.subcore_barrier()

    lane = lax.iota(jnp.int32, 16)
    z16 = jnp.zeros((16,), jnp.float32)

    def start_gathers(r, idx_v, gt_v, gd_v, sem):
        pltpu.async_copy(t_hbm.at[idx_v.at[0]], gt_v, sem)
        pltpu.async_copy(sd_hbm.at[idx_v.at[1]], gd_v, sem)

    def wait_gathers(idx_v, gt_v, gd_v, sem):
        pltpu.make_async_copy(t_hbm.at[idx_v.at[0]], gt_v, sem).wait()
        pltpu.make_async_copy(sd_hbm.at[idx_v.at[1]], gd_v, sem).wait()

    def half_round(r, r_next, guard, idx_v, dsts_v, dstq_v,
                   gt_v, gd_v, sem, semi):
        wait_gathers(idx_v, gt_v, gd_v, sem)
        @pl.when(guard)
        def _():
            pltpu.async_copy(idx_hbm.at[wid * RPW + r_next], idx_v, semi)
        for k in range(CH // 16):
            dv = idx_v[1, pl.ds(16 * k, 16)]
            dsts_v[pl.ds(16 * k, 16)] = dv
            dstq_v[pl.ds(16 * k, 16)] = lax.shift_right_logical(dv, 4)
        for k in range(CH // 16):
            dvec = dsts_v[pl.ds(16 * k, 16)]
            for t in range(16):
                e = 16 * k + t
                u = gt_v[e, pl.ds(HC, 16)] + gd_v[e, pl.ds(112, 16)]
                u = jnp.where(u >= 0.0, u, 0.2 * u)
                w = jnp.exp(u)
                for j in range(H):
                    msg_v[e, pl.ds(C * j, 16)] = (
                        gt_v[e, pl.ds(C * j, 16)] * w[j])
                gd_v[e, pl.ds(112, 16)] = z16
                m = dvec[t] & 15
                woff = jnp.where(m == 15, 112, m * H)
                mlo = jnp.where(lane < H, w, 0.0)
                fm = jnp.where(m == 15, 1.0, 0.0)
                val = mlo + (w - 2.0 * mlo) * fm
                gd_v[e, pl.ds(woff, 16)] = val
        pltpu.sync_copy(msg_v, acc.at[dsts_v], add=True)
        pltpu.sync_copy(gd_v, accb.at[dstq_v], add=True)
        @pl.when(guard)
        def _():
            pltpu.make_async_copy(idx_hbm.at[wid * RPW + r_next],
                                  idx_v, semi).wait()
            start_gathers(r_next, idx_v, gt_v, gd_v, sem)

    pltpu.sync_copy(idx_hbm.at[wid * RPW], idx_a)
    start_gathers(0, idx_a, gt_a, gd_a, sem_a)
    pltpu.sync_copy(idx_hbm.at[wid * RPW + 1], idx_b)
    start_gathers(1, idx_b, gt_b, gd_b, sem_b)

    def round_body(r2, carry):
        r = 2 * r2
        guard = r2 < RPW // 2 - 1
        half_round(r, r + 2, guard, idx_a, dsts_a, dstq_a,
                   gt_a, gd_a, sem_a, semi_a)
        half_round(r + 1, r + 3, guard, idx_b, dsts_b, dstq_b,
                   gt_b, gd_b, sem_b, semi_b)
        return carry

    lax.fori_loop(0, RPW // 2, round_body, 0)

    plsc.subcore_barrier()
    pltpu.sync_copy(acc.at[pl.ds(row0, rows_per_tile)],
                    msg_out.at[cc, pl.ds(row0, rows_per_tile)])
    pltpu.sync_copy(accb.at[pl.ds(brow0, brows_per_tile)],
                    den_out.at[cc, pl.ds(brow0, brows_per_tile)])


def _post_body(m0_ref, m1_ref, d0_ref, d1_ref, b_ref, g_ref, be_ref, o_ref):
    comb = m0_ref[...] + m1_ref[...]
    den = d0_ref[...] + d1_ref[...]
    jj = lax.broadcasted_iota(jnp.int32, (H, HC), 0)
    ll = lax.broadcasted_iota(jnp.int32, (H, HC), 1)
    expand = (ll // C == jj).astype(jnp.float32)
    denb = jnp.dot(den, expand, preferred_element_type=jnp.float32)
    o = comb / denb + b_ref[...]
    mean = jnp.mean(o, axis=0, keepdims=True)
    var = jnp.mean((o - mean) ** 2, axis=0, keepdims=True)
    o = (o - mean) * lax.rsqrt(var + 1e-5) * g_ref[...] + be_ref[...]
    o_ref[...] = jnp.where(o >= 0.0, o, 0.01 * o)


_post = pl.pallas_call(
    _post_body,
    in_specs=[
        pl.BlockSpec((N, HC), lambda: (0, 0)),
        pl.BlockSpec((N, HC), lambda: (0, 0)),
        pl.BlockSpec((N, H), lambda: (0, 0)),
        pl.BlockSpec((N, H), lambda: (0, 0)),
        pl.BlockSpec((1, HC), lambda: (0, 0)),
        pl.BlockSpec((1, HC), lambda: (0, 0)),
        pl.BlockSpec((1, HC), lambda: (0, 0)),
    ],
    out_specs=pl.BlockSpec((N, HC), lambda: (0, 0)),
    out_shape=jax.ShapeDtypeStruct((N, HC), jnp.float32),
)


def kernel(x, edge_idx, W, att_src, att_dst, bias, gamma, beta):
    e_real = edge_idx.shape[1]
    loops = jnp.arange(N, dtype=jnp.int32)
    n_pad = E_PAD - e_real - N
    src = jnp.concatenate([
        edge_idx[0].astype(jnp.int32), loops, jnp.zeros((n_pad,), jnp.int32)])
    dst = jnp.concatenate([
        edge_idx[1].astype(jnp.int32), loops, jnp.full((n_pad,), N, jnp.int32)])
    idx = jnp.stack([src.reshape(NWORK * RPW, CH),
                     dst.reshape(NWORK * RPW, CH)], axis=1)

    eye = jnp.eye(H, dtype=jnp.float32)
    ms = (att_src[:, :, None] * eye[:, None, :]).reshape(HC, H)
    md = (att_dst[:, :, None] * eye[:, None, :]).reshape(HC, H)
    bsd = jnp.concatenate(
        [ms, ms, md, md, jnp.zeros((HC, HC - 4 * H), jnp.float32)], axis=1)

    t, sd = _proj(x, W, bsd)
    za = jnp.zeros((NP, HC), jnp.float32)
    msg_p, den_p = _edge_pass(t, sd, idx, za)
    d2 = den_p.reshape(2, NB * 16, H)[:, :N]
    out = _post(msg_p[0, :N], msg_p[1, :N], d2[0], d2[1],
                bias.reshape(1, HC), gamma.reshape(1, HC), beta.reshape(1, HC))
    return out


# speedup vs baseline: 1.0662x; 1.0227x over previous
import functools

import jax
import jax.numpy as jnp
from jax import lax
from jax.experimental import pallas as pl
from jax.experimental.pallas import tpu as pltpu
from jax.experimental.pallas import tpu_sc as plsc

N = 10000
DF = 128
H = 8
C = 16
HC = H * C
NP = 10112
CH = 48
NWORK = 32
RPW = 216
E_PAD = NWORK * RPW * CH
NB = 640


def _proj_body(x_ref, w_ref, bsd_ref, t_ref, sd_ref):
    h = jnp.dot(x_ref[...], w_ref[...], preferred_element_type=jnp.float32)
    hb = jnp.dot(h, bsd_ref[...], preferred_element_type=jnp.float32)
    t_ref[:, :HC] = h
    t_ref[:, HC:] = hb
    sd_ref[:, :112] = jnp.zeros((x_ref.shape[0], 112), jnp.float32)
    sd_ref[:, 112:] = hb[:, 16:32]


_proj = pl.pallas_call(
    _proj_body,
    grid=(10,),
    in_specs=[
        pl.BlockSpec((N // 10, DF), lambda i: (i, 0)),
        pl.BlockSpec((DF, HC), lambda i: (0, 0)),
        pl.BlockSpec((DF, HC), lambda i: (0, 0)),
    ],
    out_specs=[
        pl.BlockSpec((N // 10, 2 * HC), lambda i: (i, 0)),
        pl.BlockSpec((N // 10, HC), lambda i: (i, 0)),
    ],
    out_shape=[
        jax.ShapeDtypeStruct((N, 2 * HC), jnp.float32),
        jax.ShapeDtypeStruct((N, HC), jnp.float32),
    ],
)


_mesh = plsc.VectorSubcoreMesh(core_axis_name="c", subcore_axis_name="s")


@functools.partial(
    pl.kernel,
    mesh=_mesh,
    out_type=(
        jax.ShapeDtypeStruct((2, NP, HC), jnp.float32),
        jax.ShapeDtypeStruct((2, NB, HC), jnp.float32),
    ),
    scratch_types=[
        pltpu.VMEM((2, CH), jnp.int32),
        pltpu.VMEM((2, CH), jnp.int32),
        pltpu.VMEM((CH,), jnp.int32),
        pltpu.VMEM((CH,), jnp.int32),
        pltpu.VMEM((CH,), jnp.int32),
        pltpu.VMEM((CH,), jnp.int32),
        pltpu.VMEM((CH, 2 * HC), jnp.float32),
        pltpu.VMEM((CH, HC), jnp.float32),
        pltpu.VMEM((CH, 2 * HC), jnp.float32),
        pltpu.VMEM((CH, HC), jnp.float32),
        pltpu.VMEM((CH, HC), jnp.float32),
        pltpu.VMEM_SHARED((NP, HC), jnp.float32),
        pltpu.VMEM_SHARED((NB, HC), jnp.float32),
        pltpu.SemaphoreType.DMA,
        pltpu.SemaphoreType.DMA,
        pltpu.SemaphoreType.DMA,
        pltpu.SemaphoreType.DMA,
        pltpu.SemaphoreType.DMA,
        pltpu.SemaphoreType.DMA,
    ],
)
def _edge_pass(t_hbm, sd_hbm, idx_hbm, za_hbm,
               msg_out, den_out,
               idx_a, idx_b, dsts_a, dsts_b, dstq_a, dstq_b,
               gt_a, gd_a, gt_b, gd_b, msg_v,
               acc, accb, sem_a, sem_b, semi_a, semi_b, sems_a, sems_b):
    cc = lax.axis_index("c")
    s = lax.axis_index("s")
    wid = s * 2 + cc
    rows_per_tile = NP // 16
    row0 = s * rows_per_tile
    brows_per_tile = NB // 16
    brow0 = s * brows_per_tile

    pltpu.sync_copy(za_hbm.at[pl.ds(row0, rows_per_tile)],
                    acc.at[pl.ds(row0, rows_per_tile)])
    pltpu.sync_copy(za_hbm.at[pl.ds(0, brows_per_tile)],
                    accb.at[pl.ds(brow0, brows_per_tile)])
    plsc.subcore_barrier()

    lane = lax.iota(jnp.int32, 16)
    z16 = jnp.zeros((16,), jnp.float32)

    def start_gathers(r, idx_v, gt_v, gd_v, sem):
        pltpu.async_copy(t_hbm.at[idx_v.at[0]], gt_v, sem)
        pltpu.async_copy(sd_hbm.at[idx_v.at[1]], gd_v, sem)

    def wait_gathers(idx_v, gt_v, gd_v, sem):
        pltpu.make_async_copy(t_hbm.at[idx_v.at[0]], gt_v, sem).wait()
        pltpu.make_async_copy(sd_hbm.at[idx_v.at[1]], gd_v, sem).wait()

    def half_round(r, r_next, guard, idx_v, dsts_v, dstq_v,
                   gt_v, gd_v, sem, semi, sems):
        wait_gathers(idx_v, gt_v, gd_v, sem)
        @pl.when(guard)
        def _():
            pltpu.async_copy(idx_hbm.at[wid * RPW + r_next], idx_v, semi)
        for k in range(CH // 16):
            dv = idx_v[1, pl.ds(16 * k, 16)]
            dsts_v[pl.ds(16 * k, 16)] = dv
            dstq_v[pl.ds(16 * k, 16)] = lax.shift_right_logical(dv, 4)
        for k in range(CH // 16):
            dvec = dsts_v[pl.ds(16 * k, 16)]
            for t in range(16):
                e = 16 * k + t
                u = gt_v[e, pl.ds(HC, 16)] + gd_v[e, pl.ds(112, 16)]
                u = jnp.where(u >= 0.0, u, 0.2 * u)
                w = jnp.exp(u)
                for j in range(H):
                    msg_v[e, pl.ds(C * j, 16)] = (
                        gt_v[e, pl.ds(C * j, 16)] * w[j])
                gd_v[e, pl.ds(112, 16)] = z16
                m = dvec[t] & 15
                woff = jnp.where(m == 15, 112, m * H)
                mlo = jnp.where(lane < H, w, 0.0)
                fm = jnp.where(m == 15, 1.0, 0.0)
                val = mlo + (w - 2.0 * mlo) * fm
                gd_v[e, pl.ds(woff, 16)] = val
        c1 = pltpu.async_copy(msg_v, acc.at[dsts_v], sems, add=True)
        c2 = pltpu.async_copy(gd_v, accb.at[dstq_v], sems, add=True)
        c1.wait()
        c2.wait()
        @pl.when(guard)
        def _():
            pltpu.make_async_copy(idx_hbm.at[wid * RPW + r_next],
                                  idx_v, semi).wait()
            start_gathers(r_next, idx_v, gt_v, gd_v, sem)

    pltpu.sync_copy(idx_hbm.at[wid * RPW], idx_a)
    start_gathers(0, idx_a, gt_a, gd_a, sem_a)
    pltpu.sync_copy(idx_hbm.at[wid * RPW + 1], idx_b)
    start_gathers(1, idx_b, gt_b, gd_b, sem_b)

    def round_body(r2, carry):
        r = 2 * r2
        guard = r2 < RPW // 2 - 1
        half_round(r, r + 2, guard, idx_a, dsts_a, dstq_a,
                   gt_a, gd_a, sem_a, semi_a, sems_a)
        half_round(r + 1, r + 3, guard, idx_b, dsts_b, dstq_b,
                   gt_b, gd_b, sem_b, semi_b, sems_b)
        return carry

    lax.fori_loop(0, RPW // 2, round_body, 0)

    plsc.subcore_barrier()
    pltpu.sync_copy(acc.at[pl.ds(row0, rows_per_tile)],
                    msg_out.at[cc, pl.ds(row0, rows_per_tile)])
    pltpu.sync_copy(accb.at[pl.ds(brow0, brows_per_tile)],
                    den_out.at[cc, pl.ds(brow0, brows_per_tile)])


def _post_body(m0_ref, m1_ref, d0_ref, d1_ref, b_ref, g_ref, be_ref, o_ref):
    comb = m0_ref[...] + m1_ref[...]
    den = d0_ref[...] + d1_ref[...]
    jj = lax.broadcasted_iota(jnp.int32, (H, HC), 0)
    ll = lax.broadcasted_iota(jnp.int32, (H, HC), 1)
    expand = (ll // C == jj).astype(jnp.float32)
    denb = jnp.dot(den, expand, preferred_element_type=jnp.float32)
    o = comb / denb + b_ref[...]
    mean = jnp.mean(o, axis=0, keepdims=True)
    var = jnp.mean((o - mean) ** 2, axis=0, keepdims=True)
    o = (o - mean) * lax.rsqrt(var + 1e-5) * g_ref[...] + be_ref[...]
    o_ref[...] = jnp.where(o >= 0.0, o, 0.01 * o)


_post = pl.pallas_call(
    _post_body,
    in_specs=[
        pl.BlockSpec((N, HC), lambda: (0, 0)),
        pl.BlockSpec((N, HC), lambda: (0, 0)),
        pl.BlockSpec((N, H), lambda: (0, 0)),
        pl.BlockSpec((N, H), lambda: (0, 0)),
        pl.BlockSpec((1, HC), lambda: (0, 0)),
        pl.BlockSpec((1, HC), lambda: (0, 0)),
        pl.BlockSpec((1, HC), lambda: (0, 0)),
    ],
    out_specs=pl.BlockSpec((N, HC), lambda: (0, 0)),
    out_shape=jax.ShapeDtypeStruct((N, HC), jnp.float32),
)


def kernel(x, edge_idx, W, att_src, att_dst, bias, gamma, beta):
    e_real = edge_idx.shape[1]
    loops = jnp.arange(N, dtype=jnp.int32)
    n_pad = E_PAD - e_real - N
    src = jnp.concatenate([
        edge_idx[0].astype(jnp.int32), loops, jnp.zeros((n_pad,), jnp.int32)])
    dst = jnp.concatenate([
        edge_idx[1].astype(jnp.int32), loops, jnp.full((n_pad,), N, jnp.int32)])
    idx = jnp.stack([src.reshape(NWORK * RPW, CH),
                     dst.reshape(NWORK * RPW, CH)], axis=1)

    eye = jnp.eye(H, dtype=jnp.float32)
    ms = (att_src[:, :, None] * eye[:, None, :]).reshape(HC, H)
    md = (att_dst[:, :, None] * eye[:, None, :]).reshape(HC, H)
    bsd = jnp.concatenate(
        [ms, ms, md, md, jnp.zeros((HC, HC - 4 * H), jnp.float32)], axis=1)

    t, sd = _proj(x, W, bsd)
    za = jnp.zeros((NP, HC), jnp.float32)
    msg_p, den_p = _edge_pass(t, sd, idx, za)
    d2 = den_p.reshape(2, NB * 16, H)[:, :N]
    out = _post(msg_p[0, :N], msg_p[1, :N], d2[0], d2[1],
                bias.reshape(1, HC), gamma.reshape(1, HC), beta.reshape(1, HC))
    return out


# speedup vs baseline: 1.0955x; 1.0275x over previous
import functools

import jax
import jax.numpy as jnp
from jax import lax
from jax.experimental import pallas as pl
from jax.experimental.pallas import tpu as pltpu
from jax.experimental.pallas import tpu_sc as plsc

N = 10000
DF = 128
H = 8
C = 16
HC = H * C
NP = 10112
CH = 48
NWORK = 32
RPW = 216
E_PAD = NWORK * RPW * CH
NB = 640


def _proj_body(x_ref, w_ref, bsd_ref, t_ref, sd_ref):
    h = jnp.dot(x_ref[...], w_ref[...], preferred_element_type=jnp.float32)
    hb = jnp.dot(h, bsd_ref[...], preferred_element_type=jnp.float32)
    t_ref[:, :HC] = h
    t_ref[:, HC:] = hb
    sd_ref[:, :112] = jnp.zeros((x_ref.shape[0], 112), jnp.float32)
    sd_ref[:, 112:] = hb[:, 16:32]


_proj = pl.pallas_call(
    _proj_body,
    grid=(10,),
    in_specs=[
        pl.BlockSpec((N // 10, DF), lambda i: (i, 0)),
        pl.BlockSpec((DF, HC), lambda i: (0, 0)),
        pl.BlockSpec((DF, HC), lambda i: (0, 0)),
    ],
    out_specs=[
        pl.BlockSpec((N // 10, 2 * HC), lambda i: (i, 0)),
        pl.BlockSpec((N // 10, HC), lambda i: (i, 0)),
    ],
    out_shape=[
        jax.ShapeDtypeStruct((N, 2 * HC), jnp.float32),
        jax.ShapeDtypeStruct((N, HC), jnp.float32),
    ],
)


_mesh = plsc.VectorSubcoreMesh(core_axis_name="c", subcore_axis_name="s")


@functools.partial(
    pl.kernel,
    mesh=_mesh,
    out_type=(
        jax.ShapeDtypeStruct((2, NP, HC), jnp.float32),
        jax.ShapeDtypeStruct((2, NB, HC), jnp.float32),
    ),
    scratch_types=[
        pltpu.VMEM((2, CH), jnp.int32),
        pltpu.VMEM((2, CH), jnp.int32),
        pltpu.VMEM((CH,), jnp.int32),
        pltpu.VMEM((CH,), jnp.int32),
        pltpu.VMEM((CH,), jnp.int32),
        pltpu.VMEM((CH,), jnp.int32),
        pltpu.VMEM((CH, 2 * HC), jnp.float32),
        pltpu.VMEM((CH, HC), jnp.float32),
        pltpu.VMEM((CH, 2 * HC), jnp.float32),
        pltpu.VMEM((CH, HC), jnp.float32),
        pltpu.VMEM((CH, HC), jnp.float32),
        pltpu.VMEM_SHARED((NP, HC), jnp.float32),
        pltpu.VMEM_SHARED((NB, HC), jnp.float32),
        pltpu.SemaphoreType.DMA,
        pltpu.SemaphoreType.DMA,
        pltpu.SemaphoreType.DMA,
        pltpu.SemaphoreType.DMA,
        pltpu.SemaphoreType.DMA,
        pltpu.SemaphoreType.DMA,
    ],
)
def _edge_pass(t_hbm, sd_hbm, idx_hbm, za_hbm,
               msg_out, den_out,
               idx_a, idx_b, dsts_a, dsts_b, dstq_a, dstq_b,
               gt_a, gd_a, gt_b, gd_b, msg_v,
               acc, accb, sem_a, sem_b, semi_a, semi_b, sems_a, sems_b):
    cc = lax.axis_index("c")
    s = lax.axis_index("s")
    wid = s * 2 + cc
    rows_per_tile = NP // 16
    row0 = s * rows_per_tile
    brows_per_tile = NB // 16
    brow0 = s * brows_per_tile

    pltpu.sync_copy(za_hbm.at[pl.ds(row0, rows_per_tile)],
                    acc.at[pl.ds(row0, rows_per_tile)])
    pltpu.sync_copy(za_hbm.at[pl.ds(0, brows_per_tile)],
                    accb.at[pl.ds(brow0, brows_per_tile)])
    plsc.subcore_barrier()

    lane = lax.iota(jnp.int32, 16)
    z16 = jnp.zeros((16,), jnp.float32)

    def start_gathers(r, idx_v, gt_v, gd_v, sem):
        pltpu.async_copy(t_hbm.at[idx_v.at[0]], gt_v, sem)
        pltpu.async_copy(sd_hbm.at[idx_v.at[1]], gd_v, sem)

    def wait_gathers(idx_v, gt_v, gd_v, sem):
        pltpu.make_async_copy(t_hbm.at[idx_v.at[0]], gt_v, sem).wait()
        pltpu.make_async_copy(sd_hbm.at[idx_v.at[1]], gd_v, sem).wait()

    def half_round(r, r_next, guard, idx_v, dsts_v, dstq_v,
                   gt_v, gd_v, sem, semi, sems):
        wait_gathers(idx_v, gt_v, gd_v, sem)
        @pl.when(guard)
        def _():
            pltpu.async_copy(idx_hbm.at[wid * RPW + r_next], idx_v, semi)
        for k in range(CH // 16):
            dv = idx_v[1, pl.ds(16 * k, 16)]
            dsts_v[pl.ds(16 * k, 16)] = dv
            dstq_v[pl.ds(16 * k, 16)] = lax.shift_right_logical(dv, 4)
        for k in range(CH // 16):
            dvec = dsts_v[pl.ds(16 * k, 16)]
            for t in range(16):
                e = 16 * k + t
                u = gt_v[e, pl.ds(HC, 16)] + gd_v[e, pl.ds(112, 16)]
                u = jnp.where(u >= 0.0, u, 0.2 * u)
                w = jnp.exp(u)
                for j in range(H):
                    msg_v[e, pl.ds(C * j, 16)] = (
                        gt_v[e, pl.ds(C * j, 16)] * w[j])
                gd_v[e, pl.ds(112, 16)] = z16
                m = dvec[t] & 15
                woff = jnp.where(m == 15, 112, m * H)
                mlo = jnp.where(lane < H, w, 0.0)
                fm = jnp.where(m == 15, 1.0, 0.0)
                val = mlo + (w - 2.0 * mlo) * fm
                gd_v[e, pl.ds(woff, 16)] = val
        c1 = pltpu.async_copy(msg_v, acc.at[dsts_v], sems, add=True)
        c2 = pltpu.async_copy(gd_v, accb.at[dstq_v], sems, add=True)
        c2.wait()
        @pl.when(guard)
        def _():
            pltpu.make_async_copy(idx_hbm.at[wid * RPW + r_next],
                                  idx_v, semi).wait()
            start_gathers(r_next, idx_v, gt_v, gd_v, sem)
        c1.wait()

    pltpu.sync_copy(idx_hbm.at[wid * RPW], idx_a)
    start_gathers(0, idx_a, gt_a, gd_a, sem_a)
    pltpu.sync_copy(idx_hbm.at[wid * RPW + 1], idx_b)
    start_gathers(1, idx_b, gt_b, gd_b, sem_b)

    def round_body(r2, carry):
        r = 2 * r2
        guard = r2 < RPW // 2 - 1
        half_round(r, r + 2, guard, idx_a, dsts_a, dstq_a,
                   gt_a, gd_a, sem_a, semi_a, sems_a)
        half_round(r + 1, r + 3, guard, idx_b, dsts_b, dstq_b,
                   gt_b, gd_b, sem_b, semi_b, sems_b)
        return carry

    lax.fori_loop(0, RPW // 2, round_body, 0)

    plsc.subcore_barrier()
    pltpu.sync_copy(acc.at[pl.ds(row0, rows_per_tile)],
                    msg_out.at[cc, pl.ds(row0, rows_per_tile)])
    pltpu.sync_copy(accb.at[pl.ds(brow0, brows_per_tile)],
                    den_out.at[cc, pl.ds(brow0, brows_per_tile)])


def _post_body(m0_ref, m1_ref, d0_ref, d1_ref, b_ref, g_ref, be_ref, o_ref):
    comb = m0_ref[...] + m1_ref[...]
    den = d0_ref[...] + d1_ref[...]
    jj = lax.broadcasted_iota(jnp.int32, (H, HC), 0)
    ll = lax.broadcasted_iota(jnp.int32, (H, HC), 1)
    expand = (ll // C == jj).astype(jnp.float32)
    denb = jnp.dot(den, expand, preferred_element_type=jnp.float32)
    o = comb / denb + b_ref[...]
    mean = jnp.mean(o, axis=0, keepdims=True)
    var = jnp.mean((o - mean) ** 2, axis=0, keepdims=True)
    o = (o - mean) * lax.rsqrt(var + 1e-5) * g_ref[...] + be_ref[...]
    o_ref[...] = jnp.where(o >= 0.0, o, 0.01 * o)


_post = pl.pallas_call(
    _post_body,
    in_specs=[
        pl.BlockSpec((N, HC), lambda: (0, 0)),
        pl.BlockSpec((N, HC), lambda: (0, 0)),
        pl.BlockSpec((N, H), lambda: (0, 0)),
        pl.BlockSpec((N, H), lambda: (0, 0)),
        pl.BlockSpec((1, HC), lambda: (0, 0)),
        pl.BlockSpec((1, HC), lambda: (0, 0)),
        pl.BlockSpec((1, HC), lambda: (0, 0)),
    ],
    out_specs=pl.BlockSpec((N, HC), lambda: (0, 0)),
    out_shape=jax.ShapeDtypeStruct((N, HC), jnp.float32),
)


def kernel(x, edge_idx, W, att_src, att_dst, bias, gamma, beta):
    e_real = edge_idx.shape[1]
    loops = jnp.arange(N, dtype=jnp.int32)
    n_pad = E_PAD - e_real - N
    src = jnp.concatenate([
        edge_idx[0].astype(jnp.int32), loops, jnp.zeros((n_pad,), jnp.int32)])
    dst = jnp.concatenate([
        edge_idx[1].astype(jnp.int32), loops, jnp.full((n_pad,), N, jnp.int32)])
    idx = jnp.stack([src.reshape(NWORK * RPW, CH),
                     dst.reshape(NWORK * RPW, CH)], axis=1)

    eye = jnp.eye(H, dtype=jnp.float32)
    ms = (att_src[:, :, None] * eye[:, None, :]).reshape(HC, H)
    md = (att_dst[:, :, None] * eye[:, None, :]).reshape(HC, H)
    bsd = jnp.concatenate(
        [ms, ms, md, md, jnp.zeros((HC, HC - 4 * H), jnp.float32)], axis=1)

    t, sd = _proj(x, W, bsd)
    za = jnp.zeros((NP, HC), jnp.float32)
    msg_p, den_p = _edge_pass(t, sd, idx, za)
    d2 = den_p.reshape(2, NB * 16, H)[:, :N]
    out = _post(msg_p[0, :N], msg_p[1, :N], d2[0], d2[1],
                bias.reshape(1, HC), gamma.reshape(1, HC), beta.reshape(1, HC))
    return out
